# baseline (speedup 1.0000x reference)
import functools

import jax
import jax.numpy as jnp
from jax import lax
from jax.experimental import pallas as pl
from jax.experimental.pallas import tpu as pltpu
from jax.experimental.pallas import tpu_sc as plsc

_N = 10000
_D = 128
_NPAD = 10240
_NC = 2
_NS = 16
_NW = _NC * _NS
_C = 128
_TOT = 2560
_P0 = 144
_P1 = 16
_SEG = 40
_SEGS0 = (40, 40, 40, 24)
_SEGS1 = (16,)
_NBUF = 2
_EPAD = _TOT * _C
_RPT = _NPAD // _NS
_PD = _TOT // _NW


_HR = _NPAD // 128


def _deg_body(src_hbm, dst_hbm, deg_hbm, idx_v, hist_v, iota_v, dacc):
    cid = lax.axis_index("c")
    sid = lax.axis_index("s")
    wid = cid * _NS + sid
    ones = jnp.ones((16,), jnp.float32)
    zeros = jnp.zeros((16,), jnp.float32)
    for k in range(_HR // 16):
        iota_v[0, pl.ds(k * 16, 16)] = lax.iota(jnp.int32, 16) + (k * 16)

    def zhist(r8, _):
        for rr in range(8):
            for k in range(8):
                hist_v[r8 * 8 + rr, pl.ds(k * 16, 16)] = zeros
        return 0
    lax.fori_loop(0, _HR // 8, zhist, 0)

    @pl.when(sid == 0)
    def _():
        pltpu.sync_copy(hist_v, dacc.at[0])
        pltpu.sync_copy(hist_v, dacc.at[1])
    plsc.subcore_barrier()

    for which, ind_hbm in ((0, src_hbm), (1, dst_hbm)):
        if which:
            lax.fori_loop(0, _HR // 8, zhist, 0)
        pltpu.sync_copy(ind_hbm.at[pl.ds(wid * _PD, _PD)], idx_v)

        def rbody(r, _):
            for k in range(_C // 16):
                idx = idx_v[r, pl.ds(k * 16, 16)]
                ir = lax.shift_right_logical(idx, 7)
                ic = jnp.bitwise_and(idx, 127)
                plsc.addupdate_scatter(hist_v, [ir, ic], ones)
            return 0
        lax.fori_loop(0, _PD, rbody, 0)
        tgt = dacc.at[which]
        pltpu.sync_copy(hist_v, tgt.at[iota_v.at[0]], add=True)
        plsc.subcore_barrier()

        @pl.when(sid == 0)
        def _():
            pltpu.sync_copy(dacc.at[which], deg_hbm.at[which, cid])


_deg = functools.partial(
    pl.kernel,
    out_type=jax.ShapeDtypeStruct((2, _NC, _HR, 128), jnp.float32),
    mesh=plsc.VectorSubcoreMesh(core_axis_name="c", subcore_axis_name="s"),
    compiler_params=pltpu.CompilerParams(needs_layout_passes=False),
    scratch_types=[
        pltpu.VMEM((_PD, _C), jnp.int32),
        pltpu.VMEM((_HR, 128), jnp.float32),
        pltpu.VMEM((1, _HR), jnp.int32),
        pltpu.VMEM_SHARED((2, _HR, 128), jnp.float32),
    ],
)(_deg_body)


def _agg_body(h_hbm, src_hbm, dst_hbm, out_hbm, src_v, dst_v, rows_v, accum,
              *sems):
    cid = lax.axis_index("c")
    sid = lax.axis_index("s")
    zeros = jnp.zeros((16,), jnp.float32)

    def zr(r, _):
        for k in range(_D // 16):
            rows_v[0, r, pl.ds(k * 16, 16)] = zeros
        return 0
    lax.fori_loop(0, _C, zr, 0)
    for t in range(_RPT // _C):
        pltpu.sync_copy(rows_v.at[0], accum.at[pl.ds(sid * _RPT + t * _C, _C)])
    plsc.subcore_barrier()

    bufs = tuple((rows_v.at[i], sems[i]) for i in range(_NBUF))

    def run(start, seg_lens):
        off = 0
        for seg_len in seg_lens:
            base = start + off
            pltpu.sync_copy(src_hbm.at[pl.ds(base, seg_len)],
                            src_v.at[pl.ds(0, seg_len)])
            pltpu.sync_copy(dst_hbm.at[pl.ds(base, seg_len)],
                            dst_v.at[pl.ds(0, seg_len)])
            for i, (buf, sem) in enumerate(bufs):
                pltpu.async_copy(h_hbm.at[src_v.at[i]], buf, sem)

            def step(jj, _):
                for par, (buf, sem) in enumerate(bufs):
                    j = jj * _NBUF + par
                    pltpu.make_async_copy(h_hbm.at[pl.ds(0, _C)], buf, sem).wait()
                    pltpu.sync_copy(buf, accum.at[dst_v.at[j]], add=True)
                    pltpu.async_copy(h_hbm.at[src_v.at[j + _NBUF]], buf, sem)
                return 0
            lax.fori_loop(0, seg_len // _NBUF - 1, step, 0)
            for par, (buf, sem) in enumerate(bufs):
                j = seg_len - _NBUF + par
                pltpu.make_async_copy(h_hbm.at[pl.ds(0, _C)], buf, sem).wait()
                pltpu.sync_copy(buf, accum.at[dst_v.at[j]], add=True)
            off += seg_len

    @pl.when(cid == 0)
    def _():
        run(sid * _P0, _SEGS0)

    @pl.when(cid == 1)
    def _():
        run(_NS * _P0 + sid * _P1, _SEGS1)

    plsc.subcore_barrier()
    pltpu.sync_copy(accum.at[pl.ds(sid * _RPT, _RPT)],
                    out_hbm.at[cid, pl.ds(sid * _RPT, _RPT)])


_agg = functools.partial(
    pl.kernel,
    out_type=jax.ShapeDtypeStruct((_NC, _NPAD, _D), jnp.float32),
    mesh=plsc.VectorSubcoreMesh(core_axis_name="c", subcore_axis_name="s"),
    compiler_params=pltpu.CompilerParams(needs_layout_passes=False),
    scratch_types=[
        pltpu.VMEM((_SEG, _C), jnp.int32),
        pltpu.VMEM((_SEG, _C), jnp.int32),
        pltpu.VMEM((_NBUF, _C, _D), jnp.float32),
        pltpu.VMEM_SHARED((_NPAD, _D), jnp.float32),
    ] + [pltpu.SemaphoreType.DMA] * _NBUF,
)(_agg_body)


def _scale_body(x_ref, d0_ref, d1_ref, h_ref):
    n = lax.rsqrt(d0_ref[...] + d1_ref[...] + 1.0)
    h_ref[...] = x_ref[...] * n


def _final_body(p0_ref, p1_ref, h_ref, d0_ref, d1_ref, w_ref, b_ref, o_ref):
    nd = lax.rsqrt(d0_ref[...] + d1_ref[...] + 1.0)
    s = (p0_ref[...] + p1_ref[...] + h_ref[...]) * nd
    o_ref[...] = jnp.dot(s, w_ref[...],
                         preferred_element_type=jnp.float32) + b_ref[...]


_BR = 1024
_NB = _NPAD // _BR


def kernel(x, edge_index, W, b, use_weighted_edge):
    src = edge_index[0]
    dst = edge_index[1]
    pad = _EPAD - src.shape[0]
    fill = jnp.full((pad,), _N, jnp.int32)
    src_p = jnp.concatenate([src, fill]).reshape(_TOT, _C)
    dst_p = jnp.concatenate([dst, fill]).reshape(_TOT, _C)

    deg = _deg(src_p, dst_p).reshape(2, _NC, _NPAD)
    ds0 = deg[0, 0].reshape(_NPAD, 1)
    ds1 = deg[0, 1].reshape(_NPAD, 1)
    dd0 = deg[1, 0].reshape(_NPAD, 1)
    dd1 = deg[1, 1].reshape(_NPAD, 1)

    h = pl.pallas_call(
        _scale_body,
        grid=(_NB,),
        in_specs=[pl.BlockSpec((_BR, _D), lambda j: (j, 0)),
                  pl.BlockSpec((_BR, 1), lambda j: (j, 0)),
                  pl.BlockSpec((_BR, 1), lambda j: (j, 0))],
        out_specs=pl.BlockSpec((_BR, _D), lambda j: (j, 0)),
        out_shape=jax.ShapeDtypeStruct((_NPAD, _D), jnp.float32),
    )(x, ds0, ds1)

    parts = _agg(h, src_p, dst_p).reshape(_NC * _NPAD, _D)

    out = pl.pallas_call(
        _final_body,
        grid=(_NB,),
        in_specs=[pl.BlockSpec((_BR, _D), lambda j: (j, 0)),
                  pl.BlockSpec((_BR, _D), lambda j: (j + _NB, 0)),
                  pl.BlockSpec((_BR, _D), lambda j: (j, 0)),
                  pl.BlockSpec((_BR, 1), lambda j: (j, 0)),
                  pl.BlockSpec((_BR, 1), lambda j: (j, 0)),
                  pl.BlockSpec((_D, _D), lambda j: (0, 0)),
                  pl.BlockSpec((1, _D), lambda j: (0, 0))],
        out_specs=pl.BlockSpec((_BR, _D), lambda j: (j, 0)),
        out_shape=jax.ShapeDtypeStruct((_N, _D), jnp.float32),
    )(parts, parts, h, dd0, dd1, W, b.reshape(1, _D))
    return out


# speedup vs baseline: 1.0047x; 1.0047x over previous
import functools

import jax
import jax.numpy as jnp
from jax import lax
from jax.experimental import pallas as pl
from jax.experimental.pallas import tpu as pltpu
from jax.experimental.pallas import tpu_sc as plsc

_N = 10000
_D = 128
_NPAD = 10240
_NC = 2
_NS = 16
_NW = _NC * _NS
_C = 128
_TOT = 2560
_P0 = 144
_P1 = 16
_SEG = 40
_SEGS0 = (40, 40, 40, 24)
_SEGS1 = (16,)
_NBUF = 2
_EPAD = _TOT * _C
_RPT = _NPAD // _NS
_PD = _TOT // _NW


def _deg_body(src_hbm, dst_hbm, deg_hbm, idx_v, hist_v):
    cid = lax.axis_index("c")
    sid = lax.axis_index("s")
    wid = cid * _NS + sid
    ones = jnp.ones((16,), jnp.float32)
    zeros = jnp.zeros((16,), jnp.float32)
    for which, ind_hbm in ((0, src_hbm), (1, dst_hbm)):
        def zbody(i, _):
            hist_v[pl.ds(i * 16, 16)] = zeros
            return 0
        lax.fori_loop(0, _NPAD // 16, zbody, 0)
        pltpu.sync_copy(ind_hbm.at[pl.ds(wid * _PD, _PD)], idx_v)

        def rbody(r, _):
            for k in range(_C // 16):
                idx = idx_v[r, pl.ds(k * 16, 16)]
                plsc.addupdate_scatter(hist_v, [idx], ones)
            return 0
        lax.fori_loop(0, _PD, rbody, 0)
        pltpu.sync_copy(hist_v, deg_hbm.at[which, wid])


_deg = functools.partial(
    pl.kernel,
    out_type=jax.ShapeDtypeStruct((2, _NW, _NPAD), jnp.float32),
    mesh=plsc.VectorSubcoreMesh(core_axis_name="c", subcore_axis_name="s"),
    compiler_params=pltpu.CompilerParams(needs_layout_passes=False),
    scratch_types=[
        pltpu.VMEM((_PD, _C), jnp.int32),
        pltpu.VMEM((_NPAD,), jnp.float32),
    ],
)(_deg_body)


def _agg_body(h_hbm, src_hbm, dst_hbm, out_hbm, src_v, dst_v, rows_v, accum,
              *sems):
    cid = lax.axis_index("c")
    sid = lax.axis_index("s")
    zeros = jnp.zeros((16,), jnp.float32)

    def zr(r, _):
        for k in range(_D // 16):
            rows_v[0, r, pl.ds(k * 16, 16)] = zeros
        return 0
    lax.fori_loop(0, _C, zr, 0)
    for t in range(_RPT // _C):
        pltpu.sync_copy(rows_v.at[0], accum.at[pl.ds(sid * _RPT + t * _C, _C)])
    plsc.subcore_barrier()

    bufs = tuple((rows_v.at[i], sems[i]) for i in range(_NBUF))

    def run(start, seg_lens):
        off = 0
        for seg_len in seg_lens:
            base = start + off
            pltpu.sync_copy(src_hbm.at[pl.ds(base, seg_len)],
                            src_v.at[pl.ds(0, seg_len)])
            pltpu.sync_copy(dst_hbm.at[pl.ds(base, seg_len)],
                            dst_v.at[pl.ds(0, seg_len)])
            for i, (buf, sem) in enumerate(bufs):
                pltpu.async_copy(h_hbm.at[src_v.at[i]], buf, sem)

            def step(jj, _):
                for par, (buf, sem) in enumerate(bufs):
                    j = jj * _NBUF + par
                    pltpu.make_async_copy(h_hbm.at[pl.ds(0, _C)], buf, sem).wait()
                    pltpu.sync_copy(buf, accum.at[dst_v.at[j]], add=True)
                    pltpu.async_copy(h_hbm.at[src_v.at[j + _NBUF]], buf, sem)
                return 0
            lax.fori_loop(0, seg_len // _NBUF - 1, step, 0)
            for par, (buf, sem) in enumerate(bufs):
                j = seg_len - _NBUF + par
                pltpu.make_async_copy(h_hbm.at[pl.ds(0, _C)], buf, sem).wait()
                pltpu.sync_copy(buf, accum.at[dst_v.at[j]], add=True)
            off += seg_len

    @pl.when(cid == 0)
    def _():
        run(sid * _P0, _SEGS0)

    @pl.when(cid == 1)
    def _():
        run(_NS * _P0 + sid * _P1, _SEGS1)

    plsc.subcore_barrier()
    pltpu.sync_copy(accum.at[pl.ds(sid * _RPT, _RPT)],
                    out_hbm.at[cid, pl.ds(sid * _RPT, _RPT)])


_agg = functools.partial(
    pl.kernel,
    out_type=jax.ShapeDtypeStruct((_NC, _NPAD, _D), jnp.float32),
    mesh=plsc.VectorSubcoreMesh(core_axis_name="c", subcore_axis_name="s"),
    compiler_params=pltpu.CompilerParams(needs_layout_passes=False),
    scratch_types=[
        pltpu.VMEM((_SEG, _C), jnp.int32),
        pltpu.VMEM((_SEG, _C), jnp.int32),
        pltpu.VMEM((_NBUF, _C, _D), jnp.float32),
        pltpu.VMEM_SHARED((_NPAD, _D), jnp.float32),
    ] + [pltpu.SemaphoreType.DMA] * _NBUF,
)(_agg_body)


def _norm_body(deg_ref, out_ref):
    d = deg_ref[...]
    s_src = jnp.sum(d[0:_NW, :], axis=0, keepdims=True)
    s_dst = jnp.sum(d[_NW:, :], axis=0, keepdims=True)
    s = jnp.concatenate([s_src, s_dst], axis=0) + 1.0
    out_ref[...] = lax.rsqrt(jnp.maximum(s, 1.0))


def _scale_body(x_ref, n_ref, h_ref):
    h_ref[...] = x_ref[...] * n_ref[...]


def _final_body(p0_ref, p1_ref, h_ref, nd_ref, w_ref, b_ref, o_ref):
    s = (p0_ref[...] + p1_ref[...] + h_ref[...]) * nd_ref[...]
    o_ref[...] = jnp.dot(s, w_ref[...],
                         preferred_element_type=jnp.float32) + b_ref[...]


_BR = 1024
_NB = _NPAD // _BR


def kernel(x, edge_index, W, b, use_weighted_edge):
    src = edge_index[0]
    dst = edge_index[1]
    pad = _EPAD - src.shape[0]
    fill = jnp.full((pad,), _N, jnp.int32)
    src_p = jnp.concatenate([src, fill]).reshape(_TOT, _C)
    dst_p = jnp.concatenate([dst, fill]).reshape(_TOT, _C)

    deg = _deg(src_p, dst_p)

    norms = pl.pallas_call(
        _norm_body,
        grid=(_NB,),
        in_specs=[pl.BlockSpec((2 * _NW, _BR), lambda j: (0, j))],
        out_specs=pl.BlockSpec((2, _BR), lambda j: (0, j)),
        out_shape=jax.ShapeDtypeStruct((2, _NPAD), jnp.float32),
    )(deg.reshape(2 * _NW, _NPAD))
    nsrc = norms[0].reshape(_NPAD, 1)
    ndst = norms[1].reshape(_NPAD, 1)

    h = pl.pallas_call(
        _scale_body,
        grid=(_NB,),
        in_specs=[pl.BlockSpec((_BR, _D), lambda j: (j, 0)),
                  pl.BlockSpec((_BR, 1), lambda j: (j, 0))],
        out_specs=pl.BlockSpec((_BR, _D), lambda j: (j, 0)),
        out_shape=jax.ShapeDtypeStruct((_NPAD, _D), jnp.float32),
    )(x, nsrc)

    parts = _agg(h, src_p, dst_p).reshape(_NC * _NPAD, _D)

    out = pl.pallas_call(
        _final_body,
        grid=(_NB,),
        in_specs=[pl.BlockSpec((_BR, _D), lambda j: (j, 0)),
                  pl.BlockSpec((_BR, _D), lambda j: (j + _NB, 0)),
                  pl.BlockSpec((_BR, _D), lambda j: (j, 0)),
                  pl.BlockSpec((_BR, 1), lambda j: (j, 0)),
                  pl.BlockSpec((_D, _D), lambda j: (0, 0)),
                  pl.BlockSpec((1, _D), lambda j: (0, 0))],
        out_specs=pl.BlockSpec((_BR, _D), lambda j: (j, 0)),
        out_shape=jax.ShapeDtypeStruct((_N, _D), jnp.float32),
    )(parts, parts, h, ndst, W, b.reshape(1, _D))
    return out


# speedup vs baseline: 1.0194x; 1.0146x over previous
import functools

import jax
import jax.numpy as jnp
from jax import lax
from jax.experimental import pallas as pl
from jax.experimental.pallas import tpu as pltpu
from jax.experimental.pallas import tpu_sc as plsc

_N = 10000
_D = 128
_NPAD = 10240
_NC = 2
_NS = 16
_NW = _NC * _NS
_C = 128
_TOT = 2560
_P0 = 144
_P1 = 16
_SEG = 40
_SEGS0 = (40, 40, 40, 24)
_SEGS1 = (16,)
_NBUF = 2
_EPAD = _TOT * _C
_RPT = _NPAD // _NS
_PD0 = 96
_PD1 = 64


def _deg_body(src_hbm, dst_hbm, deg_hbm, idx_v, hist_v):
    cid = lax.axis_index("c")
    sid = lax.axis_index("s")
    wid = cid * _NS + sid
    ones = jnp.ones((16,), jnp.float32)
    zeros = jnp.zeros((16,), jnp.float32)

    def rbody(r, _):
        for k in range(_C // 16):
            idx = idx_v[r, pl.ds(k * 16, 16)]
            plsc.addupdate_scatter(hist_v, [idx], ones)
        return 0

    for which, ind_hbm in ((0, src_hbm), (1, dst_hbm)):
        def zbody(i, _):
            for k in range(8):
                hist_v[pl.ds(i * 128 + k * 16, 16)] = zeros
            return 0
        lax.fori_loop(0, _NPAD // 128, zbody, 0)

        @pl.when(cid == 0)
        def _():
            pltpu.sync_copy(ind_hbm.at[pl.ds(sid * _PD0, _PD0)], idx_v)
            lax.fori_loop(0, _PD0, rbody, 0)

        @pl.when(cid == 1)
        def _():
            pltpu.sync_copy(ind_hbm.at[pl.ds(_NS * _PD0 + sid * _PD1, _PD1)],
                            idx_v.at[pl.ds(0, _PD1)])
            lax.fori_loop(0, _PD1, rbody, 0)

        pltpu.sync_copy(hist_v, deg_hbm.at[which, wid])


_deg = functools.partial(
    pl.kernel,
    out_type=jax.ShapeDtypeStruct((2, _NW, _NPAD), jnp.float32),
    mesh=plsc.VectorSubcoreMesh(core_axis_name="c", subcore_axis_name="s"),
    compiler_params=pltpu.CompilerParams(needs_layout_passes=False),
    scratch_types=[
        pltpu.VMEM((_PD0, _C), jnp.int32),
        pltpu.VMEM((_NPAD,), jnp.float32),
    ],
)(_deg_body)


def _agg_body(h_hbm, src_hbm, dst_hbm, out_hbm, src_v, dst_v, rows_v, accum,
              *sems):
    cid = lax.axis_index("c")
    sid = lax.axis_index("s")
    zeros = jnp.zeros((16,), jnp.float32)

    def zr(r, _):
        for k in range(_D // 16):
            rows_v[0, r, pl.ds(k * 16, 16)] = zeros
        return 0
    lax.fori_loop(0, _C, zr, 0)
    for t in range(_RPT // _C):
        pltpu.sync_copy(rows_v.at[0], accum.at[pl.ds(sid * _RPT + t * _C, _C)])
    plsc.subcore_barrier()

    bufs = tuple((rows_v.at[i], sems[i]) for i in range(_NBUF))

    def run(start, seg_lens):
        off = 0
        for seg_len in seg_lens:
            base = start + off
            pltpu.sync_copy(src_hbm.at[pl.ds(base, seg_len)],
                            src_v.at[pl.ds(0, seg_len)])
            pltpu.sync_copy(dst_hbm.at[pl.ds(base, seg_len)],
                            dst_v.at[pl.ds(0, seg_len)])
            for i, (buf, sem) in enumerate(bufs):
                pltpu.async_copy(h_hbm.at[src_v.at[i]], buf, sem)

            def step(jj, _):
                for par, (buf, sem) in enumerate(bufs):
                    j = jj * _NBUF + par
                    pltpu.make_async_copy(h_hbm.at[pl.ds(0, _C)], buf, sem).wait()
                    pltpu.sync_copy(buf, accum.at[dst_v.at[j]], add=True)
                    pltpu.async_copy(h_hbm.at[src_v.at[j + _NBUF]], buf, sem)
                return 0
            lax.fori_loop(0, seg_len // _NBUF - 1, step, 0)
            for par, (buf, sem) in enumerate(bufs):
                j = seg_len - _NBUF + par
                pltpu.make_async_copy(h_hbm.at[pl.ds(0, _C)], buf, sem).wait()
                pltpu.sync_copy(buf, accum.at[dst_v.at[j]], add=True)
            off += seg_len

    @pl.when(cid == 0)
    def _():
        run(sid * _P0, _SEGS0)

    @pl.when(cid == 1)
    def _():
        run(_NS * _P0 + sid * _P1, _SEGS1)

    plsc.subcore_barrier()
    pltpu.sync_copy(accum.at[pl.ds(sid * _RPT, _RPT)],
                    out_hbm.at[cid, pl.ds(sid * _RPT, _RPT)])


_agg = functools.partial(
    pl.kernel,
    out_type=jax.ShapeDtypeStruct((_NC, _NPAD, _D), jnp.float32),
    mesh=plsc.VectorSubcoreMesh(core_axis_name="c", subcore_axis_name="s"),
    compiler_params=pltpu.CompilerParams(needs_layout_passes=False),
    scratch_types=[
        pltpu.VMEM((_SEG, _C), jnp.int32),
        pltpu.VMEM((_SEG, _C), jnp.int32),
        pltpu.VMEM((_NBUF, _C, _D), jnp.float32),
        pltpu.VMEM_SHARED((_NPAD, _D), jnp.float32),
    ] + [pltpu.SemaphoreType.DMA] * _NBUF,
)(_agg_body)


def _norm_body(deg_ref, out_ref):
    d = deg_ref[...]
    s_src = jnp.sum(d[0:_NW, :], axis=0, keepdims=True)
    s_dst = jnp.sum(d[_NW:, :], axis=0, keepdims=True)
    s = jnp.concatenate([s_src, s_dst], axis=0) + 1.0
    out_ref[...] = lax.rsqrt(jnp.maximum(s, 1.0))


def _scale_body(x_ref, n_ref, h_ref):
    h_ref[...] = x_ref[...] * n_ref[...]


def _final_body(p0_ref, p1_ref, h_ref, nd_ref, w_ref, b_ref, o_ref):
    s = (p0_ref[...] + p1_ref[...] + h_ref[...]) * nd_ref[...]
    o_ref[...] = jnp.dot(s, w_ref[...],
                         preferred_element_type=jnp.float32) + b_ref[...]


_BR = 1024
_NB = _NPAD // _BR


def kernel(x, edge_index, W, b, use_weighted_edge):
    src = edge_index[0]
    dst = edge_index[1]
    pad = _EPAD - src.shape[0]
    fill = jnp.full((pad,), _N, jnp.int32)
    src_p = jnp.concatenate([src, fill]).reshape(_TOT, _C)
    dst_p = jnp.concatenate([dst, fill]).reshape(_TOT, _C)

    deg = _deg(src_p, dst_p)

    norms = pl.pallas_call(
        _norm_body,
        grid=(_NB,),
        in_specs=[pl.BlockSpec((2 * _NW, _BR), lambda j: (0, j))],
        out_specs=pl.BlockSpec((2, _BR), lambda j: (0, j)),
        out_shape=jax.ShapeDtypeStruct((2, _NPAD), jnp.float32),
    )(deg.reshape(2 * _NW, _NPAD))
    nsrc = norms[0].reshape(_NPAD, 1)
    ndst = norms[1].reshape(_NPAD, 1)

    h = pl.pallas_call(
        _scale_body,
        grid=(_NB,),
        in_specs=[pl.BlockSpec((_BR, _D), lambda j: (j, 0)),
                  pl.BlockSpec((_BR, 1), lambda j: (j, 0))],
        out_specs=pl.BlockSpec((_BR, _D), lambda j: (j, 0)),
        out_shape=jax.ShapeDtypeStruct((_NPAD, _D), jnp.float32),
    )(x, nsrc)

    parts = _agg(h, src_p, dst_p).reshape(_NC * _NPAD, _D)

    out = pl.pallas_call(
        _final_body,
        grid=(_NB,),
        in_specs=[pl.BlockSpec((_BR, _D), lambda j: (j, 0)),
                  pl.BlockSpec((_BR, _D), lambda j: (j + _NB, 0)),
                  pl.BlockSpec((_BR, _D), lambda j: (j, 0)),
                  pl.BlockSpec((_BR, 1), lambda j: (j, 0)),
                  pl.BlockSpec((_D, _D), lambda j: (0, 0)),
                  pl.BlockSpec((1, _D), lambda j: (0, 0))],
        out_specs=pl.BlockSpec((_BR, _D), lambda j: (j, 0)),
        out_shape=jax.ShapeDtypeStruct((_N, _D), jnp.float32),
    )(parts, parts, h, ndst, W, b.reshape(1, _D))
    return out
